# baseline (speedup 1.0000x reference)
import functools

import jax
import jax.numpy as jnp
from jax import lax
from jax.experimental import pallas as pl
from jax.experimental.pallas import tpu as pltpu
from jax.experimental.pallas import tpu_sc as plsc

N_EXPERTS = 8
TOP_K = 2
BLOCK_T = 4096
N_TOKENS = 32768

_NC = 2
_NS = 16
_NW = _NC * _NS
_CHUNK = N_TOKENS // _NW
_LANES = 16


def _probs_kernel(x_ref, w_ref, probs_ref):
    x = x_ref[...]
    w = w_ref[...]
    scores = jax.lax.dot_general(
        w, x, (((1,), (1,)), ((), ())), preferred_element_type=jnp.float32
    )
    m = jnp.max(scores, axis=0, keepdims=True)
    e = jnp.exp(scores - m)
    s = jnp.sum(e, axis=0, keepdims=True)
    probs_ref[...] = e / s


def _tc_probs(x, weight):
    n_tok, dim = x.shape
    n_exp = weight.shape[0]
    grid = (n_tok // BLOCK_T,)
    return pl.pallas_call(
        _probs_kernel,
        grid=grid,
        in_specs=[
            pl.BlockSpec((BLOCK_T, dim), lambda i: (i, 0)),
            pl.BlockSpec((n_exp, dim), lambda i: (0, 0)),
        ],
        out_specs=[pl.BlockSpec((n_exp, BLOCK_T), lambda i: (0, i))],
        out_shape=[jax.ShapeDtypeStruct((n_exp, n_tok), jnp.float32)],
    )(x, weight)[0]


@functools.partial(
    pl.kernel,
    mesh=plsc.VectorSubcoreMesh(core_axis_name="c", subcore_axis_name="s"),
    out_type=[
        jax.ShapeDtypeStruct((TOP_K, N_TOKENS), jnp.float32),
        jax.ShapeDtypeStruct((TOP_K, N_TOKENS), jnp.int32),
    ],
    scratch_types=[
        pltpu.VMEM((N_EXPERTS, _CHUNK), jnp.float32),
        pltpu.VMEM((TOP_K, _CHUNK), jnp.float32),
        pltpu.VMEM((TOP_K, _CHUNK), jnp.int32),
    ],
)
def _sc_top2(probs_hbm, tv_hbm, ti_hbm, p_v, tv_v, ti_v):
    wid = lax.axis_index("s") * _NC + lax.axis_index("c")
    base = wid * _CHUNK
    for e in range(N_EXPERTS):
        pltpu.sync_copy(probs_hbm.at[e, pl.ds(base, _CHUNK)], p_v.at[e])

    def body(j, _):
        sl = pl.ds(j * _LANES, _LANES)
        p = [p_v[e, sl] for e in range(N_EXPERTS)]
        v1 = p[0]
        i1 = jnp.zeros((_LANES,), jnp.int32)
        for e in range(1, N_EXPERTS):
            e_vec = jnp.full((_LANES,), e, jnp.int32)
            c = p[e] > v1
            v1 = jnp.where(c, p[e], v1)
            i1 = jnp.where(c, e_vec, i1)
        neg = jnp.full((_LANES,), -jnp.inf, jnp.float32)
        v2 = neg
        i2 = jnp.zeros((_LANES,), jnp.int32)
        for e in range(N_EXPERTS):
            e_vec = jnp.full((_LANES,), e, jnp.int32)
            cand = jnp.where(i1 == e_vec, neg, p[e])
            c = cand > v2
            v2 = jnp.where(c, cand, v2)
            i2 = jnp.where(c, e_vec, i2)
        denom = v1 + v2 + 1e-9
        tv_v[0, sl] = v1 / denom
        tv_v[1, sl] = v2 / denom
        ti_v[0, sl] = i1
        ti_v[1, sl] = i2
        return 0

    lax.fori_loop(0, _CHUNK // _LANES, body, 0)

    for k in range(TOP_K):
        pltpu.sync_copy(tv_v.at[k], tv_hbm.at[k, pl.ds(base, _CHUNK)])
        pltpu.sync_copy(ti_v.at[k], ti_hbm.at[k, pl.ds(base, _CHUNK)])


def kernel(x, weight):
    probs_t = _tc_probs(x, weight)
    tv_t, ti_t = _sc_top2(probs_t)
    return tv_t.T, ti_t.T, probs_t.T


# speedup vs baseline: 1.7247x; 1.7247x over previous
import jax
import jax.numpy as jnp
from jax.experimental import pallas as pl

N_EXPERTS = 8
TOP_K = 2
BLOCK_T = 4096


def _gate_kernel(x_ref, w_ref, probs_ref, tv_ref, ti_ref):
    x = x_ref[...]
    w = w_ref[...]
    scores = jax.lax.dot_general(
        w, x, (((1,), (1,)), ((), ())), preferred_element_type=jnp.float32
    )
    m = jnp.max(scores, axis=0, keepdims=True)
    e = jnp.exp(scores - m)
    s = jnp.sum(e, axis=0, keepdims=True)
    probs = e / s
    probs_ref[...] = probs

    v1 = jnp.max(probs, axis=0, keepdims=True)
    i1 = jnp.argmax(probs, axis=0).reshape(1, -1)
    row = jax.lax.broadcasted_iota(jnp.int32, probs.shape, 0)
    masked = jnp.where(row == i1, -jnp.inf, probs)
    v2 = jnp.max(masked, axis=0, keepdims=True)
    i2 = jnp.argmax(masked, axis=0).reshape(1, -1)
    denom = v1 + v2 + 1e-9
    tv_ref[...] = jnp.concatenate([v1 / denom, v2 / denom], axis=0)
    ti_ref[...] = jnp.concatenate([i1, i2], axis=0).astype(jnp.int32)


def kernel(x, weight):
    n_tok, dim = x.shape
    n_exp = weight.shape[0]
    grid = (n_tok // BLOCK_T,)
    probs_t, tv_t, ti_t = pl.pallas_call(
        _gate_kernel,
        grid=grid,
        in_specs=[
            pl.BlockSpec((BLOCK_T, dim), lambda i: (i, 0)),
            pl.BlockSpec((n_exp, dim), lambda i: (0, 0)),
        ],
        out_specs=[
            pl.BlockSpec((n_exp, BLOCK_T), lambda i: (0, i)),
            pl.BlockSpec((TOP_K, BLOCK_T), lambda i: (0, i)),
            pl.BlockSpec((TOP_K, BLOCK_T), lambda i: (0, i)),
        ],
        out_shape=[
            jax.ShapeDtypeStruct((n_exp, n_tok), jnp.float32),
            jax.ShapeDtypeStruct((TOP_K, n_tok), jnp.float32),
            jax.ShapeDtypeStruct((TOP_K, n_tok), jnp.int32),
        ],
    )(x, weight)
    return tv_t.T, ti_t.T, probs_t.T
